# Initial kernel scaffold; baseline (speedup 1.0000x reference)
import jax
import jax.numpy as jnp
from jax.experimental import pallas as pl


def kernel(model_output, targets, time_slices):
    raise NotImplementedError("write your pallas kernel here")


# speedup vs baseline: 7.3878x; 7.3878x over previous
import functools

import jax
import jax.numpy as jnp
from jax import lax
from jax.experimental import pallas as pl
from jax.experimental.pallas import tpu as pltpu
from jax.experimental.pallas import tpu_sc as plsc

N_ROWS = 16384
D = 128
NUM_SLICES = 8
NUM_KEYS = 2 * NUM_SLICES
PENALTY_WEIGHT = 0.1

NC = 2
NS = 16
NW = NC * NS
RPW = N_ROWS // NW
IDX_CHUNK = 128
NCHUNK = RPW // IDX_CHUNK

_mesh = plsc.VectorSubcoreMesh(
    core_axis_name="c", subcore_axis_name="s", num_cores=NC, num_subcores=NS
)


@functools.partial(
    pl.kernel,
    out_type=jax.ShapeDtypeStruct((NC, NUM_KEYS, D), jnp.float32),
    mesh=_mesh,
    scratch_types=[
        pltpu.VMEM((RPW, D), jnp.float32),
        pltpu.VMEM((RPW,), jnp.int32),
        pltpu.VMEM((RPW,), jnp.int32),
        pltpu.VMEM((NCHUNK, IDX_CHUNK), jnp.int32),
        pltpu.VMEM((NUM_KEYS, D), jnp.float32),
        pltpu.VMEM_SHARED((NUM_KEYS, D), jnp.float32),
    ],
)
def _segment_sums(mo_hbm, tg_hbm, ts_hbm, sums_out,
                  rows_v, tg_v, ts_v, keys_v, z_sums, acc_sums):
    cid = lax.axis_index("c")
    sid = lax.axis_index("s")
    wid = cid * NS + sid
    base = wid * RPW

    pltpu.sync_copy(mo_hbm.at[pl.ds(base, RPW)], rows_v)
    pltpu.sync_copy(tg_hbm.at[pl.ds(base, RPW)], tg_v)
    pltpu.sync_copy(ts_hbm.at[pl.ds(base, RPW)], ts_v)

    zero16 = jnp.zeros((16,), jnp.float32)
    for r in range(NUM_KEYS):
        for c in range(D // 16):
            z_sums[r, pl.ds(c * 16, 16)] = zero16

    for i in range(RPW // 16):
        k16 = tg_v[pl.ds(i * 16, 16)] * NUM_SLICES + ts_v[pl.ds(i * 16, 16)]
        keys_v[i // (IDX_CHUNK // 16), pl.ds((i % (IDX_CHUNK // 16)) * 16, 16)] = k16

    @pl.when(sid == 0)
    def _():
        pltpu.sync_copy(z_sums, acc_sums)

    plsc.subcore_barrier()

    for j in range(NCHUNK):
        pltpu.sync_copy(rows_v.at[pl.ds(j * IDX_CHUNK, IDX_CHUNK)],
                        acc_sums.at[keys_v.at[j]], add=True)

    plsc.subcore_barrier()

    @pl.when(sid == 0)
    def _():
        pltpu.sync_copy(acc_sums, sums_out.at[cid])


def _finalize_body(sums_ref, tg_ref, ts_ref, out_ref):
    s = sums_ref[...]
    s16 = s[0] + s[1]
    key = tg_ref[...] * NUM_SLICES + ts_ref[...]

    cnt = []
    for b in range(NUM_KEYS):
        cnt.append(jnp.sum((key == b).astype(jnp.float32)))

    n = jnp.float32(0.0)
    present = []
    for t in range(NUM_SLICES):
        p = jnp.where(cnt[t] + cnt[NUM_SLICES + t] > 0, 1.0, 0.0)
        present.append(p)
        n = n + p

    inv_n = 1.0 / jnp.maximum(n, 1.0)
    inv_nm1 = 1.0 / jnp.maximum(n - 1.0, 1.0)

    def var_mean(base_key):
        means = []
        for t in range(NUM_SLICES):
            c = cnt[base_key + t]
            inv_c = jnp.where(c > 0, 1.0 / jnp.maximum(c, 1.0), 0.0)
            means.append(s16[base_key + t:base_key + t + 1] * inv_c)
        mu = means[0] * present[0]
        for t in range(1, NUM_SLICES):
            mu = mu + means[t] * present[t]
        mu = mu * inv_n
        var = ((means[0] - mu) ** 2) * present[0]
        for t in range(1, NUM_SLICES):
            var = var + ((means[t] - mu) ** 2) * present[t]
        var = var * inv_nm1
        return jnp.mean(var, axis=1, keepdims=True)

    penalty = (var_mean(0) + var_mean(NUM_SLICES)) / 2.0
    penalty = jnp.where(n < 2, 0.0, PENALTY_WEIGHT * penalty)
    out_ref[...] = penalty


_finalize = pl.pallas_call(
    _finalize_body,
    out_shape=jax.ShapeDtypeStruct((1, 1), jnp.float32),
)


def kernel(model_output, targets, time_slices):
    tg = targets.astype(jnp.int32)
    ts = time_slices.astype(jnp.int32)
    sums = _segment_sums(model_output, tg, ts)
    out = _finalize(sums,
                    tg.reshape(N_ROWS // D, D),
                    ts.reshape(N_ROWS // D, D))
    return out[0, 0]
